# Initial kernel scaffold; baseline (speedup 1.0000x reference)
import jax
import jax.numpy as jnp
from jax.experimental import pallas as pl


def kernel(x, token_table, pos_table):
    raise NotImplementedError("write your pallas kernel here")


# speedup vs baseline: 3.1706x; 3.1706x over previous
import functools

import jax
import jax.numpy as jnp
from jax import lax
from jax.experimental import pallas as pl
from jax.experimental.pallas import tpu as pltpu
from jax.experimental.pallas import tpu_sc as plsc

VOCAB = 100000
SEQ = 200
DIM = 32
BATCH = 1024

NROWS = BATCH * SEQ
NW = 32
ROWS_PER_W = NROWS // NW
CHUNK = 128
NCHUNK = ROWS_PER_W // CHUNK
POSF = SEQ * DIM


def _body(x_hbm, tok_hbm, pos_hbm, out_hbm, idx_v, pos_v, buf, gsem, osem):
    wid = lax.axis_index("s") * 2 + lax.axis_index("c")
    base = wid * ROWS_PER_W

    pltpu.sync_copy(x_hbm.at[wid], idx_v)
    pltpu.sync_copy(pos_hbm, pos_v)

    def start_gather(c, b):
        return pltpu.async_copy(tok_hbm.at[idx_v.at[c]], buf.at[b], gsem)

    def start_store(c, b):
        return pltpu.async_copy(
            buf.at[b], out_hbm.at[pl.ds(base + c * CHUNK, CHUNK)], osem)

    gathers = {0: start_gather(0, 0)}
    stores = {}
    for c in range(NCHUNK):
        b = c % 2
        if c + 1 < NCHUNK:
            if c >= 1:
                stores[c - 1].wait()
            gathers[c + 1] = start_gather(c + 1, 1 - b)
        gathers[c].wait()

        pbase = (c * CHUNK * DIM) % POSF
        bufc = buf.at[b]

        def add_pos(r, _, pbase=pbase, bufc=bufc):
            o = lax.rem(pbase + r * DIM, POSF)
            bufc[r, 0:16] = bufc[r, 0:16] + pos_v[pl.ds(o, 16)]
            bufc[r, 16:32] = bufc[r, 16:32] + pos_v[pl.ds(o + 16, 16)]
            return 0

        lax.fori_loop(0, CHUNK, add_pos, 0)
        stores[c] = start_store(c, b)
    stores[NCHUNK - 2].wait()
    stores[NCHUNK - 1].wait()


@functools.partial(jax.jit, static_argnames=())
def kernel(x, token_table, pos_table):
    x_w = x.reshape(NW, NCHUNK, CHUNK).astype(jnp.int32)
    pos_flat = pos_table.reshape(POSF)
    run = pl.kernel(
        _body,
        out_type=jax.ShapeDtypeStruct((NROWS, DIM), jnp.float32),
        mesh=plsc.VectorSubcoreMesh(core_axis_name="c", subcore_axis_name="s"),
        scratch_types=[
            pltpu.VMEM((NCHUNK, CHUNK), jnp.int32),
            pltpu.VMEM((POSF,), jnp.float32),
            pltpu.VMEM((2, CHUNK, DIM), jnp.float32),
            pltpu.SemaphoreType.DMA,
            pltpu.SemaphoreType.DMA,
        ],
        compiler_params=pltpu.CompilerParams(use_tc_tiling_on_sc=False),
    )
    out = run(x_w, token_table, pos_flat)
    return out.reshape(BATCH, SEQ, DIM)
